# Initial kernel scaffold; baseline (speedup 1.0000x reference)
import jax
import jax.numpy as jnp
from jax.experimental import pallas as pl


def kernel(edge_index, edge_w, W1, W2, n_nodes):
    raise NotImplementedError("write your pallas kernel here")


# speedup vs baseline: 2.9495x; 2.9495x over previous
import functools

import jax
import jax.numpy as jnp
from jax import lax
from jax.experimental import pallas as pl
from jax.experimental.pallas import tpu as pltpu
from jax.experimental.pallas import tpu_sc as plsc

N = 4096
E = 65536
IN_C = 4
OUT_C = 2

NC = 2
NS = 16
L = 16

EPT = E // NS
STRIPE_ROWS = 64
STRIPE_WORDS = STRIPE_ROWS * N
N_STRIPES = N // STRIPE_ROWS
JOBS = OUT_C * N_STRIPES
JOBS_PER_CORE = JOBS // NC
TILE_WORDS = STRIPE_WORDS // NS
MAT_WORDS = N * N


def _densify_body(src_h, dst_h, ew_h, f_h, rnk_h, np_h, outa_h, outb_h,
                  srcs, dsts, ews, fv, fv2, rnk, vals, mva, mvb, idxb,
                  zbuf, sha, shb):
  c = lax.axis_index("c")
  s = lax.axis_index("s")
  base = s * EPT

  pltpu.sync_copy(rnk_h.at[pl.ds(base, EPT)], rnk)
  pltpu.sync_copy(src_h.at[pl.ds(base, EPT)], srcs)
  pltpu.sync_copy(dst_h.at[pl.ds(base, EPT)], dsts)
  for j in range(IN_C):
    pltpu.sync_copy(ew_h.at[j, pl.ds(base, EPT)], ews.at[j])
  pltpu.sync_copy(f_h, fv)
  pltpu.sync_copy(np_h, fv2)

  fvec = fv[pl.ds(0, L)]

  def _pre(k, _):
    off = k * L
    sv = jnp.minimum(srcs[pl.ds(off, L)], N - 1)
    dv = jnp.minimum(dsts[pl.ds(off, L)], N - 1)
    srcs[pl.ds(off, L)] = sv * N + dv
    e = [ews[j, pl.ds(off, L)] for j in range(IN_C)]
    for m in range(2 * OUT_C):
      v = fvec[m * IN_C] * e[0]
      for j in range(1, IN_C):
        v = v + fvec[m * IN_C + j] * e[j]
      vals[m, pl.ds(off, L)] = v
    return 0

  lax.fori_loop(0, EPT // L, _pre, 0)

  def _z(k, _):
    zbuf[pl.ds(k * L, L)] = jnp.zeros((L,), jnp.float32)
    return 0

  lax.fori_loop(0, TILE_WORDS // L, _z, 0)

  gmin = srcs[pl.ds(0, L)][0]
  gmax = srcs[pl.ds(EPT - L, L)][L - 1]
  npass = fv2[pl.ds(0, L)][0]

  def _round(r, _):
    job = NC * r + c
    i = job // N_STRIPES
    stripe = job % N_STRIPES
    lo = stripe * STRIPE_WORDS

    pltpu.sync_copy(zbuf, sha.at[pl.ds(s * TILE_WORDS, TILE_WORDS)])
    pltpu.sync_copy(zbuf, shb.at[pl.ds(s * TILE_WORDS, TILE_WORDS)])
    plsc.subcore_barrier()

    active = (gmax >= lo) & (gmin < lo + STRIPE_WORDS)

    @pl.when(active)
    def _scatter():
      def _pass(p, _):
        def _bld(kb, _):
          off = kb * L
          g = srcs[pl.ds(off, L)]
          local = g - lo
          m = ((local >= 0) & (local < STRIPE_WORDS)
               & (rnk[pl.ds(off, L)] == p))
          idxb[pl.ds(off, L)] = jnp.where(m, local, 0)
          va = jnp.where(i == 0, vals[0, pl.ds(off, L)],
                         vals[1, pl.ds(off, L)])
          vb = jnp.where(i == 0, vals[2, pl.ds(off, L)],
                         vals[3, pl.ds(off, L)])
          zero = jnp.zeros((L,), jnp.float32)
          mva[pl.ds(off, L)] = jnp.where(m, va, zero)
          mvb[pl.ds(off, L)] = jnp.where(m, vb, zero)
          return 0

        lax.fori_loop(0, EPT // L, _bld, 0)
        pltpu.sync_copy(mva, sha.at[idxb], add=True)
        pltpu.sync_copy(mvb, shb.at[idxb], add=True)
        return 0

      lax.fori_loop(0, npass, _pass, 0)

    plsc.subcore_barrier()

    pos = i * MAT_WORDS + stripe * STRIPE_WORDS + s * TILE_WORDS
    pltpu.sync_copy(sha.at[pl.ds(s * TILE_WORDS, TILE_WORDS)],
                    outa_h.at[pl.ds(pos, TILE_WORDS)])
    pltpu.sync_copy(shb.at[pl.ds(s * TILE_WORDS, TILE_WORDS)],
                    outb_h.at[pl.ds(pos, TILE_WORDS)])
    return 0

  lax.fori_loop(0, JOBS_PER_CORE, _round, 0)


def _densify(src, dst, ew, fcat, rank, npass_arr):
  mesh = plsc.VectorSubcoreMesh(core_axis_name="c", subcore_axis_name="s")
  out_t = (jax.ShapeDtypeStruct((OUT_C * N * N,), jnp.float32),
           jax.ShapeDtypeStruct((OUT_C * N * N,), jnp.float32))
  scratch = [
      pltpu.VMEM((EPT,), jnp.int32),
      pltpu.VMEM((EPT,), jnp.int32),
      pltpu.VMEM((IN_C, EPT), jnp.float32),
      pltpu.VMEM((L,), jnp.float32),
      pltpu.VMEM((L,), jnp.int32),
      pltpu.VMEM((EPT,), jnp.int32),
      pltpu.VMEM((2 * OUT_C, EPT), jnp.float32),
      pltpu.VMEM((EPT,), jnp.float32),
      pltpu.VMEM((EPT,), jnp.float32),
      pltpu.VMEM((EPT,), jnp.int32),
      pltpu.VMEM((TILE_WORDS,), jnp.float32),
      pltpu.VMEM_SHARED((STRIPE_WORDS,), jnp.float32),
      pltpu.VMEM_SHARED((STRIPE_WORDS,), jnp.float32),
  ]
  k = pl.kernel(_densify_body, out_type=out_t, mesh=mesh,
                scratch_types=scratch)
  return k(src, dst, ew, fcat, rank, npass_arr)


def _mm_body(a_ref, b_ref, o_ref, acc_ref):
  @pl.when(pl.program_id(3) == 0)
  def _init():
    acc_ref[...] = jnp.zeros_like(acc_ref)

  a = a_ref[0].astype(jnp.bfloat16)
  b = b_ref[0].astype(jnp.bfloat16)
  acc_ref[...] += jnp.dot(a, b, preferred_element_type=jnp.float32)

  @pl.when(pl.program_id(3) == pl.num_programs(3) - 1)
  def _out():
    o_ref[0] = acc_ref[...]


def _matmul(ad, bd, bm=1024, bn=1024, bk=2048):
  return pl.pallas_call(
      _mm_body,
      out_shape=jax.ShapeDtypeStruct((OUT_C, N, N), jnp.float32),
      grid=(OUT_C, N // bm, N // bn, N // bk),
      in_specs=[
          pl.BlockSpec((1, bm, bk), lambda i, m, n, k: (i, m, k)),
          pl.BlockSpec((1, bk, bn), lambda i, m, n, k: (i, k, n)),
      ],
      out_specs=pl.BlockSpec((1, bm, bn), lambda i, m, n, k: (i, m, n)),
      scratch_shapes=[pltpu.VMEM((bm, bn), jnp.float32)],
      compiler_params=pltpu.CompilerParams(
          dimension_semantics=("parallel", "parallel", "parallel",
                               "arbitrary")),
  )(ad, bd)


def kernel(edge_index, edge_w, W1, W2, n_nodes):
  src = edge_index[0].astype(jnp.int32)
  dst = edge_index[1].astype(jnp.int32)
  ew = edge_w.astype(jnp.float32)
  f1 = jax.nn.softmax(W1.astype(jnp.float32), axis=1)
  f2 = jax.nn.softmax(W2.astype(jnp.float32), axis=1)
  fcat = jnp.concatenate([f1.reshape(-1), f2.reshape(-1)])
  gidx = (jnp.minimum(src, n_nodes - 1) * n_nodes
          + jnp.minimum(dst, n_nodes - 1))
  order = jnp.argsort(gidx).astype(jnp.int32)
  src_s = jnp.take(src, order)
  dst_s = jnp.take(dst, order)
  ew_s = jnp.take(ew, order, axis=1)
  gs = jnp.take(gidx, order)
  ar = jnp.arange(E, dtype=jnp.int32)
  is_start = jnp.concatenate(
      [jnp.ones((1,), bool), gs[1:] != gs[:-1]])
  first = jnp.where(is_start, ar, 0)
  rank = (ar - lax.cummax(first)).astype(jnp.int32)
  npass_arr = jnp.full((L,), jnp.max(rank) + 1, jnp.int32)
  ad_flat, bd_flat = _densify(src_s, dst_s, ew_s, fcat, rank, npass_arr)
  ad = ad_flat.reshape(OUT_C, N, N)
  bd = bd_flat.reshape(OUT_C, N, N)
  h = _matmul(ad, bd)
  return h, lax.stop_gradient(f1), lax.stop_gradient(f2)


# speedup vs baseline: 4.6038x; 1.5609x over previous
import functools

import jax
import jax.numpy as jnp
from jax import lax
from jax.experimental import pallas as pl
from jax.experimental.pallas import tpu as pltpu
from jax.experimental.pallas import tpu_sc as plsc

N = 4096
E = 65536
IN_C = 4
OUT_C = 2

NC = 2
NS = 16
L = 16

NW = NC * NS
ROWS_W = N // NW
CROWS = 4
NCHUNK = ROWS_W // CROWS
CWORDS = CROWS * N
CAP = 4096
MAT_WORDS = N * N


def _densify_body(gs_h, ew_h, f_h, rnk_h, bnd_h, np_h, outa_h, outb_h,
                  fv, fv2, bndv, gsv, rkv, e0, e1, e2, e3, vv,
                  b0, b1, b2, b3):
  ews = (e0, e1, e2, e3)
  bufs = (b0, b1, b2, b3)
  c = lax.axis_index("c")
  s = lax.axis_index("s")
  w = s * NC + c

  pltpu.sync_copy(f_h, fv)
  pltpu.sync_copy(np_h, fv2)
  pltpu.sync_copy(bnd_h, bndv)
  fvec = fv[pl.ds(0, L)]
  npass = fv2[pl.ds(0, L)][0]
  bv = bndv[pl.ds(pl.multiple_of(w * L, L), L)]
  my_lo = bv[0]
  my_hi = bv[1]
  blo = pl.multiple_of(my_lo - lax.rem(my_lo, 8), 8)
  nb = (my_hi - blo + CAP - 1) // CAP

  def _chunk(k, _):
    def _z(j, _):
      z = jnp.zeros((L,), jnp.float32)
      for m in range(2 * OUT_C):
        bufs[m][pl.ds(j * L, L)] = z
      return 0

    lax.fori_loop(0, CWORDS // L + 1, _z, 0)
    base = (w * ROWS_W + k * CROWS) * N

    def _batch(b, _):
      off = pl.multiple_of(blo + b * CAP, 8)
      pltpu.sync_copy(gs_h.at[pl.ds(off, CAP)], gsv)
      pltpu.sync_copy(rnk_h.at[pl.ds(off, CAP)], rkv)
      for j in range(IN_C):
        pltpu.sync_copy(
            ew_h.at[pl.ds(pl.multiple_of(j * (E + CAP) + off, 8), CAP)],
            ews[j])

      def _cmb(t, _):
        o = t * L
        e = [ews[j][pl.ds(o, L)] for j in range(IN_C)]
        for m in range(2 * OUT_C):
          v = fvec[m * IN_C] * e[0]
          for j in range(1, IN_C):
            v = v + fvec[m * IN_C + j] * e[j]
          vv[m, pl.ds(o, L)] = v
        return 0

      lax.fori_loop(0, CAP // L, _cmb, 0)

      def _pass(p, _):
        def _scan(t, _):
          o = t * L
          local = gsv[pl.ds(o, L)] - base
          m = ((local >= 0) & (local < CWORDS)
               & (rkv[pl.ds(o, L)] == p))
          idx = jnp.where(m, local, CWORDS)
          zero = jnp.zeros((L,), jnp.float32)
          for mm in range(2 * OUT_C):
            cur = plsc.load_gather(bufs[mm], [idx])
            upd = cur + jnp.where(m, vv[mm, pl.ds(o, L)], zero)
            plsc.store_scatter(bufs[mm], [idx], upd)
          return 0

        lax.fori_loop(0, CAP // L, _scan, 0)
        return 0

      lax.fori_loop(0, npass, _pass, 0)
      return 0

    lax.fori_loop(0, nb, _batch, 0)

    for mm in range(OUT_C):
      pltpu.sync_copy(bufs[mm].at[pl.ds(0, CWORDS)],
                      outa_h.at[pl.ds(mm * MAT_WORDS + base, CWORDS)])
      pltpu.sync_copy(bufs[OUT_C + mm].at[pl.ds(0, CWORDS)],
                      outb_h.at[pl.ds(mm * MAT_WORDS + base, CWORDS)])
    return 0

  lax.fori_loop(0, NCHUNK, _chunk, 0)


def _densify(gs, ew, fcat, rank, bnd, npass_arr):
  mesh = plsc.VectorSubcoreMesh(core_axis_name="c", subcore_axis_name="s")
  out_t = (jax.ShapeDtypeStruct((OUT_C * N * N,), jnp.float32),
           jax.ShapeDtypeStruct((OUT_C * N * N,), jnp.float32))
  scratch = [
      pltpu.VMEM((L,), jnp.float32),
      pltpu.VMEM((L,), jnp.int32),
      pltpu.VMEM((NW * L,), jnp.int32),
      pltpu.VMEM((CAP,), jnp.int32),
      pltpu.VMEM((CAP,), jnp.int32),
      pltpu.VMEM((CAP,), jnp.float32),
      pltpu.VMEM((CAP,), jnp.float32),
      pltpu.VMEM((CAP,), jnp.float32),
      pltpu.VMEM((CAP,), jnp.float32),
      pltpu.VMEM((2 * OUT_C, CAP), jnp.float32),
      pltpu.VMEM((CWORDS + L,), jnp.float32),
      pltpu.VMEM((CWORDS + L,), jnp.float32),
      pltpu.VMEM((CWORDS + L,), jnp.float32),
      pltpu.VMEM((CWORDS + L,), jnp.float32),
  ]
  k = pl.kernel(_densify_body, out_type=out_t, mesh=mesh,
                scratch_types=scratch,
                compiler_params=pltpu.CompilerParams(
                    needs_layout_passes=False))
  return k(gs, ew, fcat, rank, bnd, npass_arr)


def _mm_body(a_ref, b_ref, o_ref, acc_ref):
  @pl.when(pl.program_id(3) == 0)
  def _init():
    acc_ref[...] = jnp.zeros_like(acc_ref)

  a = a_ref[0].astype(jnp.bfloat16)
  b = b_ref[0].astype(jnp.bfloat16)
  acc_ref[...] += jnp.dot(a, b, preferred_element_type=jnp.float32)

  @pl.when(pl.program_id(3) == pl.num_programs(3) - 1)
  def _out():
    o_ref[0] = acc_ref[...]


def _matmul(ad, bd, bm=1024, bn=1024, bk=2048):
  return pl.pallas_call(
      _mm_body,
      out_shape=jax.ShapeDtypeStruct((OUT_C, N, N), jnp.float32),
      grid=(OUT_C, N // bm, N // bn, N // bk),
      in_specs=[
          pl.BlockSpec((1, bm, bk), lambda i, m, n, k: (i, m, k)),
          pl.BlockSpec((1, bk, bn), lambda i, m, n, k: (i, k, n)),
      ],
      out_specs=pl.BlockSpec((1, bm, bn), lambda i, m, n, k: (i, m, n)),
      scratch_shapes=[pltpu.VMEM((bm, bn), jnp.float32)],
      compiler_params=pltpu.CompilerParams(
          dimension_semantics=("parallel", "parallel", "parallel",
                               "arbitrary")),
  )(ad, bd)


def kernel(edge_index, edge_w, W1, W2, n_nodes):
  src = edge_index[0].astype(jnp.int32)
  dst = edge_index[1].astype(jnp.int32)
  ew = edge_w.astype(jnp.float32)
  f1 = jax.nn.softmax(W1.astype(jnp.float32), axis=1)
  f2 = jax.nn.softmax(W2.astype(jnp.float32), axis=1)
  fcat = jnp.concatenate([f1.reshape(-1), f2.reshape(-1)])
  gidx = (jnp.minimum(src, n_nodes - 1) * n_nodes
          + jnp.minimum(dst, n_nodes - 1))
  order = jnp.argsort(gidx).astype(jnp.int32)
  ew_s = jnp.take(ew, order, axis=1)
  gs = jnp.take(gidx, order)
  ar = jnp.arange(E, dtype=jnp.int32)
  is_start = jnp.concatenate(
      [jnp.ones((1,), bool), gs[1:] != gs[:-1]])
  first = jnp.where(is_start, ar, 0)
  rank = (ar - lax.cummax(first)).astype(jnp.int32)
  npass_arr = jnp.full((L,), jnp.max(rank) + 1, jnp.int32)
  bnd = jnp.searchsorted(
      gs, jnp.arange(NW + 1, dtype=jnp.int32) * (ROWS_W * N)).astype(jnp.int32)
  bnd_p = jnp.stack(
      [bnd[:NW], bnd[1:]] + [jnp.zeros((NW,), jnp.int32)] * (L - 2),
      axis=1).reshape(-1)
  gs_p = jnp.concatenate([gs, jnp.full((CAP,), 2**30, jnp.int32)])
  rank_p = jnp.concatenate([rank, jnp.zeros((CAP,), jnp.int32)])
  ew_p = jnp.concatenate(
      [ew_s, jnp.zeros((IN_C, CAP), jnp.float32)], axis=1).reshape(-1)
  ad_flat, bd_flat = _densify(gs_p, ew_p, fcat, rank_p, bnd_p, npass_arr)
  ad = ad_flat.reshape(OUT_C, N, N)
  bd = bd_flat.reshape(OUT_C, N, N)
  h = _matmul(ad, bd)
  return h, lax.stop_gradient(f1), lax.stop_gradient(f2)


# speedup vs baseline: 5.5935x; 1.2150x over previous
import functools

import jax
import jax.numpy as jnp
from jax import lax
from jax.experimental import pallas as pl
from jax.experimental.pallas import tpu as pltpu
from jax.experimental.pallas import tpu_sc as plsc

N = 4096
E = 65536
IN_C = 4
OUT_C = 2

NC = 2
NS = 16
L = 16

NW = NC * NS
ROWS_W = N // NW
CROWS = 4
NCHUNK = ROWS_W // CROWS
CWORDS = CROWS * N
CAP = 4096
MAT_WORDS = N * N


def _densify_body(gs_h, ew_h, f_h, rnk_h, bnd_h, outa_h, outb_h,
                  fv, bndv, gsv, rkv, e0, e1, e2, e3, vv,
                  b0, b1, b2, b3):
  ews = (e0, e1, e2, e3)
  bufs = (b0, b1, b2, b3)
  c = lax.axis_index("c")
  s = lax.axis_index("s")
  w = s * NC + c

  pltpu.sync_copy(f_h, fv)
  pltpu.sync_copy(bnd_h, bndv)
  fvec = fv[pl.ds(0, L)]
  bv = bndv[pl.ds(pl.multiple_of(w * L, L), L)]
  my_lo = bv[0]
  my_hi = bv[1]
  npass = bv[2]
  blo = pl.multiple_of(my_lo - lax.rem(my_lo, 8), 8)
  nb = (my_hi - blo + CAP - 1) // CAP

  def _stage(b):
    off = pl.multiple_of(blo + b * CAP, 8)
    pltpu.sync_copy(gs_h.at[pl.ds(off, CAP)], gsv)
    pltpu.sync_copy(rnk_h.at[pl.ds(off, CAP)], rkv)
    for j in range(IN_C):
      pltpu.sync_copy(
          ew_h.at[pl.ds(pl.multiple_of(j * (E + CAP) + off, 8), CAP)],
          ews[j])

    def _cmb(t, _):
      o = t * L
      e = [ews[j][pl.ds(o, L)] for j in range(IN_C)]
      for m in range(2 * OUT_C):
        v = fvec[m * IN_C] * e[0]
        for j in range(1, IN_C):
          v = v + fvec[m * IN_C + j] * e[j]
        vv[m, pl.ds(o, L)] = v
      return 0

    lax.fori_loop(0, CAP // L, _cmb, 0)

  def _scatter(base):
    def _pass(p, _):
      def _scan(t, _):
        o = t * L
        local = gsv[pl.ds(o, L)] - base
        m = ((local >= 0) & (local < CWORDS)
             & (rkv[pl.ds(o, L)] == p))
        idx = jnp.where(m, local, CWORDS)
        zero = jnp.zeros((L,), jnp.float32)
        for mm in range(2 * OUT_C):
          cur = plsc.load_gather(bufs[mm], [idx])
          upd = cur + jnp.where(m, vv[mm, pl.ds(o, L)], zero)
          plsc.store_scatter(bufs[mm], [idx], upd)
        return 0

      lax.fori_loop(0, CAP // L, _scan, 0)
      return 0

    lax.fori_loop(0, npass, _pass, 0)

  _stage(0)

  def _chunk(k, _):
    def _z(j, _):
      z = jnp.zeros((L,), jnp.float32)
      for m in range(2 * OUT_C):
        bufs[m][pl.ds(j * 2 * L, L)] = z
        bufs[m][pl.ds(j * 2 * L + L, L)] = z
      return 0

    lax.fori_loop(0, CWORDS // (2 * L), _z, 0)
    z16 = jnp.zeros((L,), jnp.float32)
    for m in range(2 * OUT_C):
      bufs[m][pl.ds(CWORDS, L)] = z16
    base = (w * ROWS_W + k * CROWS) * N

    @pl.when(nb > 1)
    def _restage():
      _stage(0)

    _scatter(base)

    def _over(b, _):
      _stage(b)
      _scatter(base)
      return 0

    lax.fori_loop(1, nb, _over, 0)

    for mm in range(OUT_C):
      pltpu.sync_copy(bufs[mm].at[pl.ds(0, CWORDS)],
                      outa_h.at[pl.ds(mm * MAT_WORDS + base, CWORDS)])
      pltpu.sync_copy(bufs[OUT_C + mm].at[pl.ds(0, CWORDS)],
                      outb_h.at[pl.ds(mm * MAT_WORDS + base, CWORDS)])
    return 0

  lax.fori_loop(0, NCHUNK, _chunk, 0)


def _densify(gs, ew, fcat, rank, bnd):
  mesh = plsc.VectorSubcoreMesh(core_axis_name="c", subcore_axis_name="s")
  out_t = (jax.ShapeDtypeStruct((OUT_C * N * N,), jnp.float32),
           jax.ShapeDtypeStruct((OUT_C * N * N,), jnp.float32))
  scratch = [
      pltpu.VMEM((L,), jnp.float32),
      pltpu.VMEM((NW * L,), jnp.int32),
      pltpu.VMEM((CAP,), jnp.int32),
      pltpu.VMEM((CAP,), jnp.int32),
      pltpu.VMEM((CAP,), jnp.float32),
      pltpu.VMEM((CAP,), jnp.float32),
      pltpu.VMEM((CAP,), jnp.float32),
      pltpu.VMEM((CAP,), jnp.float32),
      pltpu.VMEM((2 * OUT_C, CAP), jnp.float32),
      pltpu.VMEM((CWORDS + L,), jnp.float32),
      pltpu.VMEM((CWORDS + L,), jnp.float32),
      pltpu.VMEM((CWORDS + L,), jnp.float32),
      pltpu.VMEM((CWORDS + L,), jnp.float32),
  ]
  k = pl.kernel(_densify_body, out_type=out_t, mesh=mesh,
                scratch_types=scratch,
                compiler_params=pltpu.CompilerParams(
                    needs_layout_passes=False))
  return k(gs, ew, fcat, rank, bnd)


def _mm_body(a_ref, b_ref, o_ref):
  @pl.when(pl.program_id(3) == 0)
  def _init():
    o_ref[0] = jnp.zeros_like(o_ref[0])

  a = a_ref[0].astype(jnp.bfloat16)
  b = b_ref[0].astype(jnp.bfloat16)
  o_ref[0] += jnp.dot(a, b, preferred_element_type=jnp.float32)


def _matmul(ad, bd, bm=2048, bn=2048, bk=512):
  return pl.pallas_call(
      _mm_body,
      out_shape=jax.ShapeDtypeStruct((OUT_C, N, N), jnp.float32),
      grid=(OUT_C, N // bm, N // bn, N // bk),
      in_specs=[
          pl.BlockSpec((1, bm, bk), lambda i, m, n, k: (i, m, k)),
          pl.BlockSpec((1, bk, bn), lambda i, m, n, k: (i, k, n)),
      ],
      out_specs=pl.BlockSpec((1, bm, bn), lambda i, m, n, k: (i, m, n)),
      compiler_params=pltpu.CompilerParams(
          dimension_semantics=("parallel", "parallel", "parallel",
                               "arbitrary")),
  )(ad, bd)


def kernel(edge_index, edge_w, W1, W2, n_nodes):
  src = edge_index[0].astype(jnp.int32)
  dst = edge_index[1].astype(jnp.int32)
  ew = edge_w.astype(jnp.float32)
  f1 = jax.nn.softmax(W1.astype(jnp.float32), axis=1)
  f2 = jax.nn.softmax(W2.astype(jnp.float32), axis=1)
  fcat = jnp.concatenate([f1.reshape(-1), f2.reshape(-1)])
  gidx = (jnp.minimum(src, n_nodes - 1) * n_nodes
          + jnp.minimum(dst, n_nodes - 1))
  order = jnp.argsort(gidx).astype(jnp.int32)
  ew_s = jnp.take(ew, order, axis=1)
  gs = jnp.take(gidx, order)
  ar = jnp.arange(E, dtype=jnp.int32)
  is_start = jnp.concatenate(
      [jnp.ones((1,), bool), gs[1:] != gs[:-1]])
  first = jnp.where(is_start, ar, 0)
  rank = (ar - lax.cummax(first)).astype(jnp.int32)
  bnd = jnp.searchsorted(
      gs, jnp.arange(NW + 1, dtype=jnp.int32) * (ROWS_W * N)).astype(jnp.int32)
  w_of_e = gs // (ROWS_W * N)
  npass_w = jax.ops.segment_max(rank, w_of_e, num_segments=NW,
                                indices_are_sorted=True) + 1
  npass_w = jnp.maximum(npass_w, 1).astype(jnp.int32)
  bnd_p = jnp.stack(
      [bnd[:NW], bnd[1:], npass_w] + [jnp.zeros((NW,), jnp.int32)] * (L - 3),
      axis=1).reshape(-1)
  gs_p = jnp.concatenate([gs, jnp.full((CAP,), 2**30, jnp.int32)])
  rank_p = jnp.concatenate([rank, jnp.zeros((CAP,), jnp.int32)])
  ew_p = jnp.concatenate(
      [ew_s, jnp.zeros((IN_C, CAP), jnp.float32)], axis=1).reshape(-1)
  ad_flat, bd_flat = _densify(gs_p, ew_p, fcat, rank_p, bnd_p)
  ad = ad_flat.reshape(OUT_C, N, N)
  bd = bd_flat.reshape(OUT_C, N, N)
  h = _matmul(ad, bd)
  return h, lax.stop_gradient(f1), lax.stop_gradient(f2)


# speedup vs baseline: 8.2810x; 1.4805x over previous
import functools

import jax
import jax.numpy as jnp
from jax import lax
from jax.experimental import pallas as pl
from jax.experimental.pallas import tpu as pltpu
from jax.experimental.pallas import tpu_sc as plsc

N = 4096
E = 65536
IN_C = 4
OUT_C = 2

NC = 2
NS = 16
L = 16

NW = NC * NS
ROWS_W = N // NW
CROWS = 4
NCHUNK = ROWS_W // CROWS
CWORDS = CROWS * N
CAP = 4096
MAT_WORDS = N * N


def _densify_body(gs_h, ew_h, f_h, rnk_h, bnd_h, outa_h, outb_h,
                  fv, bndv, gsv, rkv, e0, e1, e2, e3, vv,
                  b0, b1, b2, b3):
  ews = (e0, e1, e2, e3)
  bufs = (b0, b1, b2, b3)
  c = lax.axis_index("c")
  s = lax.axis_index("s")
  w = s * NC + c

  pltpu.sync_copy(f_h, fv)
  pltpu.sync_copy(bnd_h, bndv)
  fvec = fv[pl.ds(0, L)]
  bv = bndv[pl.ds(pl.multiple_of(w * L, L), L)]
  my_lo = bv[0]
  my_hi = bv[1]
  npass = bv[2]
  blo = pl.multiple_of(my_lo - lax.rem(my_lo, 8), 8)
  nb = (my_hi - blo + CAP - 1) // CAP

  def _stage(b):
    off = pl.multiple_of(blo + b * CAP, 8)
    pltpu.sync_copy(gs_h.at[pl.ds(off, CAP)], gsv)
    pltpu.sync_copy(rnk_h.at[pl.ds(off, CAP)], rkv)
    for j in range(IN_C):
      pltpu.sync_copy(
          ew_h.at[pl.ds(pl.multiple_of(j * (E + CAP) + off, 8), CAP)],
          ews[j])

    def _cmb(t, _):
      o = t * L
      e = [ews[j][pl.ds(o, L)] for j in range(IN_C)]
      for m in range(2 * OUT_C):
        v = fvec[m * IN_C] * e[0]
        for j in range(1, IN_C):
          v = v + fvec[m * IN_C + j] * e[j]
        vv[m, pl.ds(o, L)] = v
      return 0

    lax.fori_loop(0, CAP // L, _cmb, 0)

  def _scatter(base, t0):
    hi = base + CWORDS

    def _cond(t):
      o = pl.multiple_of(t * L, L)
      return (t < CAP // L) & (gsv[pl.ds(o, L)][0] < hi)

    def _body(t):
      o = pl.multiple_of(t * L, L)
      local = gsv[pl.ds(o, L)] - base
      inc = (local >= 0) & (local < CWORDS)
      rk = rkv[pl.ds(o, L)]
      zero = jnp.zeros((L,), jnp.float32)

      def _pass(p, _):
        m = inc & (rk == p)
        idx = jnp.where(m, local, CWORDS)
        for mm in range(2 * OUT_C):
          cur = plsc.load_gather(bufs[mm], [idx])
          upd = cur + jnp.where(m, vv[mm, pl.ds(o, L)], zero)
          plsc.store_scatter(bufs[mm], [idx], upd)
        return 0

      lax.fori_loop(0, npass, _pass, 0)
      return t + 1

    stop = lax.while_loop(_cond, _body, t0)
    return jnp.maximum(stop - 1, 0)

  _stage(0)

  def _chunk(k, vstart):
    def _z(j, _):
      z = jnp.zeros((L,), jnp.float32)
      for m in range(2 * OUT_C):
        bufs[m][pl.ds(j * 2 * L, L)] = z
        bufs[m][pl.ds(j * 2 * L + L, L)] = z
      return 0

    lax.fori_loop(0, CWORDS // (2 * L), _z, 0)
    z16 = jnp.zeros((L,), jnp.float32)
    for m in range(2 * OUT_C):
      bufs[m][pl.ds(CWORDS, L)] = z16
    base = (w * ROWS_W + k * CROWS) * N

    @pl.when(nb > 1)
    def _restage():
      _stage(0)

    t0 = jnp.where(nb > 1, 0, vstart)
    vnext = _scatter(base, t0)

    def _over(b, _):
      _stage(b)
      _scatter(base, 0)
      return 0

    lax.fori_loop(1, nb, _over, 0)

    for mm in range(OUT_C):
      pltpu.sync_copy(bufs[mm].at[pl.ds(0, CWORDS)],
                      outa_h.at[pl.ds(mm * MAT_WORDS + base, CWORDS)])
      pltpu.sync_copy(bufs[OUT_C + mm].at[pl.ds(0, CWORDS)],
                      outb_h.at[pl.ds(mm * MAT_WORDS + base, CWORDS)])
    return vnext

  lax.fori_loop(0, NCHUNK, _chunk, jnp.int32(0))


def _densify(gs, ew, fcat, rank, bnd):
  mesh = plsc.VectorSubcoreMesh(core_axis_name="c", subcore_axis_name="s")
  out_t = (jax.ShapeDtypeStruct((OUT_C * N * N,), jnp.float32),
           jax.ShapeDtypeStruct((OUT_C * N * N,), jnp.float32))
  scratch = [
      pltpu.VMEM((L,), jnp.float32),
      pltpu.VMEM((NW * L,), jnp.int32),
      pltpu.VMEM((CAP,), jnp.int32),
      pltpu.VMEM((CAP,), jnp.int32),
      pltpu.VMEM((CAP,), jnp.float32),
      pltpu.VMEM((CAP,), jnp.float32),
      pltpu.VMEM((CAP,), jnp.float32),
      pltpu.VMEM((CAP,), jnp.float32),
      pltpu.VMEM((2 * OUT_C, CAP), jnp.float32),
      pltpu.VMEM((CWORDS + L,), jnp.float32),
      pltpu.VMEM((CWORDS + L,), jnp.float32),
      pltpu.VMEM((CWORDS + L,), jnp.float32),
      pltpu.VMEM((CWORDS + L,), jnp.float32),
  ]
  k = pl.kernel(_densify_body, out_type=out_t, mesh=mesh,
                scratch_types=scratch,
                compiler_params=pltpu.CompilerParams(
                    needs_layout_passes=False))
  return k(gs, ew, fcat, rank, bnd)


def _mm_body(a_ref, b_ref, o_ref):
  @pl.when(pl.program_id(3) == 0)
  def _init():
    o_ref[0] = jnp.zeros_like(o_ref[0])

  a = a_ref[0].astype(jnp.bfloat16)
  b = b_ref[0].astype(jnp.bfloat16)
  o_ref[0] += jnp.dot(a, b, preferred_element_type=jnp.float32)


def _matmul(ad, bd, bm=2048, bn=2048, bk=512):
  return pl.pallas_call(
      _mm_body,
      out_shape=jax.ShapeDtypeStruct((OUT_C, N, N), jnp.float32),
      grid=(OUT_C, N // bm, N // bn, N // bk),
      in_specs=[
          pl.BlockSpec((1, bm, bk), lambda i, m, n, k: (i, m, k)),
          pl.BlockSpec((1, bk, bn), lambda i, m, n, k: (i, k, n)),
      ],
      out_specs=pl.BlockSpec((1, bm, bn), lambda i, m, n, k: (i, m, n)),
      compiler_params=pltpu.CompilerParams(
          dimension_semantics=("parallel", "parallel", "parallel",
                               "arbitrary")),
  )(ad, bd)


def kernel(edge_index, edge_w, W1, W2, n_nodes):
  src = edge_index[0].astype(jnp.int32)
  dst = edge_index[1].astype(jnp.int32)
  ew = edge_w.astype(jnp.float32)
  f1 = jax.nn.softmax(W1.astype(jnp.float32), axis=1)
  f2 = jax.nn.softmax(W2.astype(jnp.float32), axis=1)
  fcat = jnp.concatenate([f1.reshape(-1), f2.reshape(-1)])
  gidx = (jnp.minimum(src, n_nodes - 1) * n_nodes
          + jnp.minimum(dst, n_nodes - 1))
  order = jnp.argsort(gidx).astype(jnp.int32)
  ew_s = jnp.take(ew, order, axis=1)
  gs = jnp.take(gidx, order)
  ar = jnp.arange(E, dtype=jnp.int32)
  is_start = jnp.concatenate(
      [jnp.ones((1,), bool), gs[1:] != gs[:-1]])
  first = jnp.where(is_start, ar, 0)
  rank = (ar - lax.cummax(first)).astype(jnp.int32)
  bnd = jnp.searchsorted(
      gs, jnp.arange(NW + 1, dtype=jnp.int32) * (ROWS_W * N)).astype(jnp.int32)
  w_of_e = gs // (ROWS_W * N)
  npass_w = jax.ops.segment_max(rank, w_of_e, num_segments=NW,
                                indices_are_sorted=True) + 1
  npass_w = jnp.maximum(npass_w, 1).astype(jnp.int32)
  bnd_p = jnp.stack(
      [bnd[:NW], bnd[1:], npass_w] + [jnp.zeros((NW,), jnp.int32)] * (L - 3),
      axis=1).reshape(-1)
  gs_p = jnp.concatenate([gs, jnp.full((CAP,), 2**30, jnp.int32)])
  rank_p = jnp.concatenate([rank, jnp.zeros((CAP,), jnp.int32)])
  ew_p = jnp.concatenate(
      [ew_s, jnp.zeros((IN_C, CAP), jnp.float32)], axis=1).reshape(-1)
  ad_flat, bd_flat = _densify(gs_p, ew_p, fcat, rank_p, bnd_p)
  ad = ad_flat.reshape(OUT_C, N, N)
  bd = bd_flat.reshape(OUT_C, N, N)
  h = _matmul(ad, bd)
  return h, lax.stop_gradient(f1), lax.stop_gradient(f2)
